# baseline (speedup 1.0000x reference)
import jax
import jax.numpy as jnp
from jax import lax
from jax.experimental import pallas as pl
from jax.experimental.pallas import tpu as pltpu
from jax.experimental.pallas import tpu_sc as plsc

N = 10000
D = 128
HD = 64
E = 320000
NC = 2
NS = 16
K = 128
EPT = 20480
NCHUNK = EPT // K
EPAD = EPT * NS
TILE_STRIDE = 624
TILE_ROWS = 640


NBUF = 4
ER = 8


def _lap_body(a_hbm, src_hbm, dst_hbm, val_hbm, out_hbm, *scr):
    sidxr, didxr, valsr, zbuf, acc, tbl = scr[:6]
    rows = scr[6:6 + NBUF]
    gsem = scr[6 + NBUF:6 + 2 * NBUF]
    ssem = scr[6 + 2 * NBUF:6 + 3 * NBUF]
    esem = scr[6 + 3 * NBUF:6 + 3 * NBUF + 4]
    zsem = scr[6 + 3 * NBUF + 4]
    tsem = scr[6 + 3 * NBUF + 5]
    c = lax.axis_index("c")
    s = lax.axis_index("s")

    pltpu.async_copy(a_hbm.at[c, pl.ds(s * TILE_STRIDE, TILE_ROWS)],
                     tbl.at[pl.ds(s * TILE_STRIDE, TILE_ROWS)], tsem)

    def eissue(j, sl):
        sm = esem[sl % 4]
        pltpu.async_copy(src_hbm.at[s, j], sidxr.at[sl], sm)
        pltpu.async_copy(dst_hbm.at[s, j], didxr.at[sl], sm)
        pltpu.async_copy(val_hbm.at[s, j], valsr.at[sl], sm)

    def ewait(j, sl):
        sm = esem[sl % 4]
        pltpu.make_async_copy(src_hbm.at[s, j], sidxr.at[sl], sm).wait()
        pltpu.make_async_copy(dst_hbm.at[s, j], didxr.at[sl], sm).wait()
        pltpu.make_async_copy(val_hbm.at[s, j], valsr.at[sl], sm).wait()

    def gissue(sl, b):
        pltpu.async_copy(tbl.at[sidxr.at[sl]], rows[b], gsem[b])

    def gwait(sl, b):
        pltpu.make_async_copy(tbl.at[sidxr.at[sl]], rows[b],
                              gsem[b]).wait()

    def sissue(sl, b):
        pltpu.async_copy(rows[b], acc.at[didxr.at[sl]], ssem[b], add=True)

    def swait(sl, b):
        pltpu.make_async_copy(rows[b], acc.at[didxr.at[sl]],
                              ssem[b]).wait()

    def scale(sl, b):
        rb = rows[b]

        def group(g, carry2):
            vch = valsr[sl, pl.ds(g * 16, 16)]
            for t in range(16):
                v = vch.at[jnp.full((16,), t, jnp.int32)].get(
                    mode="promise_in_bounds")
                r = g * 16 + t
                for k in range(HD // 16):
                    rb[r, pl.ds(k * 16, 16)] = rb[r, pl.ds(k * 16, 16)] * v
            return carry2

        lax.fori_loop(0, K // 16, group, 0)

    zero = jnp.zeros((16,), jnp.float32)

    def zrow(i, carry):
        for k in range(HD // 16):
            zbuf[i, pl.ds(k * 16, 16)] = zero
        return carry

    lax.fori_loop(0, 64, zrow, 0)
    for t in range(10):
        pltpu.async_copy(zbuf, acc.at[pl.ds(s * TILE_STRIDE + t * 64, 64)],
                         zsem)
    for j in range(3):
        eissue(j, j)
    for t in range(10):
        pltpu.make_async_copy(
            zbuf, acc.at[pl.ds(s * TILE_STRIDE + t * 64, 64)], zsem).wait()
    pltpu.make_async_copy(a_hbm.at[c, pl.ds(s * TILE_STRIDE, TILE_ROWS)],
                          tbl.at[pl.ds(s * TILE_STRIDE, TILE_ROWS)],
                          tsem).wait()
    plsc.subcore_barrier()

    ewait(0, 0)
    gissue(0, 0)
    ewait(1, 1)
    gissue(1, 1)

    def step(j, b8, first, do_g, do_e):
        b = b8 % NBUF
        gwait(b8, b)
        nb = (b + 2) % NBUF
        if not first:
            swait((b8 + 6) % ER, nb)
        if do_g:
            ewait(j + 2, (b8 + 2) % ER)
            gissue((b8 + 2) % ER, nb)
        if do_e:
            eissue(j + 3, (b8 + 3) % ER)
        scale(b8, b)
        sissue(b8, b)

    for b8 in range(ER):
        step(b8, b8, first=(b8 < 2), do_g=True, do_e=True)

    def middle(g, carry):
        j0 = g * ER
        for b8 in range(ER):
            step(j0 + b8, b8, first=False, do_g=True, do_e=True)
        return carry

    lax.fori_loop(1, NCHUNK // ER - 1, middle, 0)

    j0 = NCHUNK - ER
    for b8 in range(ER):
        j = j0 + b8
        step(j, b8, first=False,
             do_g=(j + 2 < NCHUNK), do_e=(j + 3 < NCHUNK))
    swait((NCHUNK - 2) % ER, (NCHUNK - 2) % NBUF)
    swait((NCHUNK - 1) % ER, (NCHUNK - 1) % NBUF)

    plsc.subcore_barrier()

    pltpu.sync_copy(acc.at[pl.ds(s * TILE_STRIDE, TILE_ROWS)],
                    out_hbm.at[c, pl.ds(s * TILE_STRIDE, TILE_ROWS)])


@jax.jit
def _lap_sc(a_split, src3, dst3, val3):
    mesh = plsc.VectorSubcoreMesh(core_axis_name="c", subcore_axis_name="s")
    f = pl.kernel(
        _lap_body,
        out_type=jax.ShapeDtypeStruct((NC, N, HD), jnp.float32),
        mesh=mesh,
        scratch_types=(
            [
                pltpu.VMEM((ER, K), jnp.int32),
                pltpu.VMEM((ER, K), jnp.int32),
                pltpu.VMEM((ER, K), jnp.float32),
                pltpu.VMEM((64, HD), jnp.float32),
                pltpu.VMEM_SHARED((N, HD), jnp.float32),
                pltpu.VMEM_SHARED((N, HD), jnp.float32),
            ]
            + [pltpu.VMEM((K, HD), jnp.float32)] * NBUF
            + [pltpu.SemaphoreType.DMA] * (2 * NBUF + 6)
        ),
        compiler_params=pltpu.CompilerParams(use_tc_tiling_on_sc=False),
    )
    return f(a_split, src3, dst3, val3)


NB = 10
R = N // NB


def _elu(x):
    return jnp.where(x > 0, x, jnp.exp(jnp.minimum(x, 0.0)) - 1.0)


def _bn_consts(stats_ref, g_ref, be_ref):
    mean = stats_ref[0:1, :] * (1.0 / N)
    ex2 = stats_ref[1:2, :] * (1.0 / N)
    var = ex2 - mean * mean
    sc = g_ref[0:1, :] * jax.lax.rsqrt(var + 1e-5)
    sh = be_ref[0:1, :] - mean * sc
    return sc, sh


def _acc_stats(stats_ref, blk, i):
    @pl.when(i == 0)
    def _():
        stats_ref[...] = jnp.zeros_like(stats_ref)

    stats_ref[0:1, :] += jnp.sum(blk, axis=0, keepdims=True)
    stats_ref[1:2, :] += jnp.sum(blk * blk, axis=0, keepdims=True)


def _pre_body(inp_ref, w_ref, b_ref, x_ref, a_ref, stats_ref):
    x = jnp.dot(inp_ref[...], w_ref[...],
                preferred_element_type=jnp.float32) + b_ref[0:1, :]
    x_ref[...] = x
    a = _elu(x)
    a_ref[...] = a
    stats_ref[...] = jnp.zeros_like(stats_ref)
    stats_ref[0:1, :] = jnp.sum(a, axis=0, keepdims=True)
    stats_ref[1:2, :] = jnp.sum(a * a, axis=0, keepdims=True)


@jax.jit
def _pre(inp, w1, b1):
    return pl.pallas_call(
        _pre_body,
        out_shape=[
            jax.ShapeDtypeStruct((N, D), jnp.float32),
            jax.ShapeDtypeStruct((N, D), jnp.float32),
            jax.ShapeDtypeStruct((8, D), jnp.float32),
        ],
    )(inp, w1, b1)


def _make_lap_step(with_res):
    def body(*refs):
        if with_res:
            (a_ref, p_ref, res_ref, stats_ref, gt, gb, bet, beb,
             wt, wb, bv, x_ref, a_out, stats_out, opst) = refs
        else:
            (a_ref, p_ref, stats_ref, gt, gb, bet, beb,
             wt, wb, bv, x_ref, a_out, stats_out, opst) = refs
        ph = pl.program_id(0)
        i = pl.program_id(1)
        op = jnp.concatenate([p_ref[0], p_ref[1]], axis=1)

        @pl.when(ph == 0)
        def _():
            _acc_stats(opst, op, i)
            @pl.when(i == 0)
            def _():
                x_ref[...] = jnp.zeros_like(x_ref)
                a_out[...] = jnp.zeros_like(a_out)
                stats_out[...] = jnp.zeros_like(stats_out)

        @pl.when(ph == 1)
        def _():
            sct, sht = _bn_consts(stats_ref, gt, bet)
            scb, shb = _bn_consts(opst, gb, beb)
            y = jnp.dot(a_ref[...] * sct + sht, wt[...],
                        preferred_element_type=jnp.float32)
            y += jnp.dot(op * scb + shb, wb[...],
                         preferred_element_type=jnp.float32)
            y += bv[0:1, :]
            if with_res:
                y += res_ref[...]
            x_ref[...] = y
            a = _elu(y)
            a_out[...] = a
            _acc_stats(stats_out, a, i)

    return body


def _make_lap_call(with_res):
    body = _make_lap_step(with_res)
    zero2 = lambda ph, i: (0, 0)
    mov2 = lambda ph, i: (i * ph, 0)
    res_spec = [pl.BlockSpec((R, D), mov2)] if with_res else []

    @jax.jit
    def call(a, p, res, stats, gt, gb, bet, beb, wt, wb, bv):
        args = [a, p] + ([res] if with_res else []) + [
            stats, gt, gb, bet, beb, wt, wb, bv]
        return pl.pallas_call(
            body,
            grid=(2, NB),
            in_specs=[
                pl.BlockSpec((R, D), mov2),
                pl.BlockSpec((NC, R, HD), lambda ph, i: (0, i, 0)),
            ] + res_spec + [
                pl.BlockSpec((8, D), zero2),
                pl.BlockSpec((1, D), zero2),
                pl.BlockSpec((1, D), zero2),
                pl.BlockSpec((1, D), zero2),
                pl.BlockSpec((1, D), zero2),
                pl.BlockSpec((D, D), zero2),
                pl.BlockSpec((D, D), zero2),
                pl.BlockSpec((1, D), zero2),
            ],
            out_specs=[
                pl.BlockSpec((R, D), mov2),
                pl.BlockSpec((R, D), mov2),
                pl.BlockSpec((8, D), zero2),
            ],
            out_shape=[
                jax.ShapeDtypeStruct((N, D), jnp.float32),
                jax.ShapeDtypeStruct((N, D), jnp.float32),
                jax.ShapeDtypeStruct((8, D), jnp.float32),
            ],
            scratch_shapes=[pltpu.VMEM((8, D), jnp.float32)],
        )(*args)

    return call


def _make_avg_step(with_res):
    def body(*refs):
        if with_res:
            (a_ref, res_ref, stats_ref, gt, bet, beb,
             wt, wb, bv, x_ref, a_out, stats_out) = refs
        else:
            (a_ref, stats_ref, gt, bet, beb,
             wt, wb, bv, x_ref, a_out, stats_out) = refs
        i = pl.program_id(0)
        sct, sht = _bn_consts(stats_ref, gt, bet)
        y = jnp.dot(a_ref[...] * sct + sht, wt[...],
                    preferred_element_type=jnp.float32)
        y += jnp.dot(beb[0:1, :], wb[...],
                     preferred_element_type=jnp.float32)
        y += bv[0:1, :]
        if with_res:
            y += res_ref[...]
        x_ref[...] = y
        a = _elu(y)
        a_out[...] = a
        _acc_stats(stats_out, a, i)

    return body


def _make_avg_call(with_res):
    body = _make_avg_step(with_res)
    zero1 = lambda i: (0, 0)
    mov1 = lambda i: (i, 0)
    res_spec = [pl.BlockSpec((R, D), mov1)] if with_res else []

    @jax.jit
    def call(a, res, stats, gt, bet, beb, wt, wb, bv):
        args = [a] + ([res] if with_res else []) + [
            stats, gt, bet, beb, wt, wb, bv]
        return pl.pallas_call(
            body,
            grid=(NB,),
            in_specs=[
                pl.BlockSpec((R, D), mov1),
            ] + res_spec + [
                pl.BlockSpec((8, D), zero1),
                pl.BlockSpec((1, D), zero1),
                pl.BlockSpec((1, D), zero1),
                pl.BlockSpec((1, D), zero1),
                pl.BlockSpec((D, D), zero1),
                pl.BlockSpec((D, D), zero1),
                pl.BlockSpec((1, D), zero1),
            ],
            out_specs=[
                pl.BlockSpec((R, D), mov1),
                pl.BlockSpec((R, D), mov1),
                pl.BlockSpec((8, D), zero1),
            ],
            out_shape=[
                jax.ShapeDtypeStruct((N, D), jnp.float32),
                jax.ShapeDtypeStruct((N, D), jnp.float32),
                jax.ShapeDtypeStruct((8, D), jnp.float32),
            ],
        )(*args)

    return call


_lap_step_nores = _make_lap_call(False)
_lap_step_res = _make_lap_call(True)
_avg_step_nores = _make_avg_call(False)
_avg_step_res = _make_avg_call(True)


def _final_body(a_ref, stats_ref, g_ref, be_ref, w_ref, b_ref, rep_ref,
                out_ref):
    sc, sh = _bn_consts(stats_ref, g_ref, be_ref)
    y = jnp.dot(a_ref[...] * sc + sh, w_ref[...],
                preferred_element_type=jnp.float32)
    out_ref[...] = y + b_ref[0:1, :] + rep_ref[...]


@jax.jit
def _final(a, stats, g, be, w2, b2, rep):
    return pl.pallas_call(
        _final_body,
        out_shape=jax.ShapeDtypeStruct((N, 120), jnp.float32),
    )(a, stats, g, be, w2, b2, rep)


def kernel(inputs, mask, L_indices, L_values, W1, b1,
           rn_g0, rn_be0, rn_W0, rn_b0,
           rn_g1, rn_be1, rn_W1, rn_b1,
           g2, be2, W2, b2):
    dst = L_indices[0].astype(jnp.int32)
    src = L_indices[1].astype(jnp.int32)
    val = L_values.astype(jnp.float32)
    dst, src, val = jax.lax.sort((dst, src, val), num_keys=1)
    pad = EPAD - E
    dst3 = jnp.pad(dst, (0, pad)).reshape(NS, NCHUNK, K)
    src3 = jnp.pad(src, (0, pad)).reshape(NS, NCHUNK, K)
    val3 = jnp.pad(val, (0, pad)).reshape(NS, NCHUNK, K)

    inp = inputs.reshape(N, 6)
    x, a, stats = _pre(inp, W1, b1.reshape(1, D))

    res = x
    for i in range(15):
        for j in range(2):
            g = (rn_g0, rn_be0, rn_W0, rn_b0) if j == 0 else \
                (rn_g1, rn_be1, rn_W1, rn_b1)
            gam, bet, w, bb = (t[i] for t in g)
            gt = gam[:D].reshape(1, D)
            gb = gam[D:].reshape(1, D)
            bt = bet[:D].reshape(1, D)
            bbt = bet[D:].reshape(1, D)
            wt = w[:D, :]
            wb = w[D:, :]
            bv = bb.reshape(1, D)
            r = res if j == 1 else None
            if i % 2 == 0:
                a_split = a.reshape(N, NC, HD).transpose(1, 0, 2)
                p = _lap_sc(a_split, src3, dst3, val3)
                if j == 1:
                    x, a, stats = _lap_step_res(
                        a, p, r, stats, gt, gb, bt, bbt, wt, wb, bv)
                else:
                    x, a, stats = _lap_step_nores(
                        a, p, None, stats, gt, gb, bt, bbt, wt, wb, bv)
            else:
                if j == 1:
                    x, a, stats = _avg_step_res(
                        a, r, stats, gt, bt, bbt, wt, wb, bv)
                else:
                    x, a, stats = _avg_step_nores(
                        a, None, stats, gt, bt, bbt, wt, wb, bv)
            if j == 1:
                res = x
    rep = jnp.tile(inp[:, 3:6], (1, 40))
    out = _final(a, stats, g2.reshape(1, D), be2.reshape(1, D), W2,
                 b2.reshape(1, 120), rep)
    return out[None]


# speedup vs baseline: 1.3367x; 1.3367x over previous
import jax
import jax.numpy as jnp
import numpy as np
from jax import lax
from jax.experimental import pallas as pl
from jax.experimental.pallas import tpu as pltpu
from jax.experimental.pallas import tpu_sc as plsc

N = 10000
D = 128
HD = 64
E = 320000
NC = 2
NS = 16
K = 128
EPT = 20480
NCHUNK = EPT // K
EPAD = EPT * NS
TILE_STRIDE = 624
TILE_ROWS = 640


NBUF = 4
ER = 8


def _lap_body(a_hbm, src_hbm, dst_hbm, val_hbm, out_hbm, *scr):
    sidxr, didxr, valsr, zbuf, acc, tbl = scr[:6]
    rows = scr[6:6 + NBUF]
    frows = scr[6 + NBUF:6 + 2 * NBUF]
    gsem = scr[6 + 2 * NBUF:6 + 3 * NBUF]
    ssem = scr[6 + 3 * NBUF:6 + 4 * NBUF]
    esem = scr[6 + 4 * NBUF:6 + 4 * NBUF + 4]
    zsem = scr[6 + 4 * NBUF + 4]
    tsem = scr[6 + 4 * NBUF + 5]
    c = lax.axis_index("c")
    s = lax.axis_index("s")

    pltpu.async_copy(a_hbm.at[c, pl.ds(s * TILE_STRIDE, TILE_ROWS)],
                     tbl.at[pl.ds(s * TILE_STRIDE, TILE_ROWS)], tsem)

    def eissue(j, sl):
        sm = esem[sl % 4]
        pltpu.async_copy(src_hbm.at[s, j], sidxr.at[sl], sm)
        pltpu.async_copy(dst_hbm.at[s, j], didxr.at[sl], sm)
        pltpu.async_copy(val_hbm.at[s, j], valsr.at[sl], sm)

    def ewait(j, sl):
        sm = esem[sl % 4]
        pltpu.make_async_copy(src_hbm.at[s, j], sidxr.at[sl], sm).wait()
        pltpu.make_async_copy(dst_hbm.at[s, j], didxr.at[sl], sm).wait()
        pltpu.make_async_copy(val_hbm.at[s, j], valsr.at[sl], sm).wait()

    def gissue(sl, b):
        pltpu.async_copy(tbl.at[sidxr.at[sl]], rows[b], gsem[b])

    def gwait(sl, b):
        pltpu.make_async_copy(tbl.at[sidxr.at[sl]], rows[b],
                              gsem[b]).wait()

    def sissue(sl, b):
        pltpu.async_copy(frows[b], acc.at[didxr.at[sl]], ssem[b], add=True)

    def swait(sl, b):
        pltpu.make_async_copy(frows[b], acc.at[didxr.at[sl]],
                              ssem[b]).wait()

    def scale(sl, b):
        rb = rows[b]
        fb = frows[b]

        def group(g, carry2):
            vch = valsr[sl, pl.ds(g * 16, 16)]
            for t in range(16):
                v = vch.at[jnp.full((16,), t, jnp.int32)].get(
                    mode="promise_in_bounds")
                r = g * 16 + t
                for k in range(HD // 32):
                    v32 = rb[r, pl.ds(k * 32, 32)]
                    ua, ub = plsc.unpack(
                        v32, format=plsc.PackFormat.INTERLEAVED)
                    fb[r, pl.ds(k * 32, 16)] = ua * v
                    fb[r, pl.ds(k * 32 + 16, 16)] = ub * v
            return carry2

        lax.fori_loop(0, K // 16, group, 0)

    zero = jnp.zeros((16,), jnp.float32)

    def zrow(i, carry):
        for k in range(HD // 16):
            zbuf[i, pl.ds(k * 16, 16)] = zero
        return carry

    lax.fori_loop(0, 64, zrow, 0)
    for t in range(10):
        pltpu.async_copy(zbuf, acc.at[pl.ds(s * TILE_STRIDE + t * 64, 64)],
                         zsem)
    for j in range(3):
        eissue(j, j)
    for t in range(10):
        pltpu.make_async_copy(
            zbuf, acc.at[pl.ds(s * TILE_STRIDE + t * 64, 64)], zsem).wait()
    pltpu.make_async_copy(a_hbm.at[c, pl.ds(s * TILE_STRIDE, TILE_ROWS)],
                          tbl.at[pl.ds(s * TILE_STRIDE, TILE_ROWS)],
                          tsem).wait()
    plsc.subcore_barrier()

    ewait(0, 0)
    gissue(0, 0)
    ewait(1, 1)
    gissue(1, 1)

    def step(j, b8, first, do_g, do_e):
        b = b8 % NBUF
        gwait(b8, b)
        nb = (b + 2) % NBUF
        if not first:
            swait((b8 + 6) % ER, nb)
        if do_g:
            ewait(j + 2, (b8 + 2) % ER)
            gissue((b8 + 2) % ER, nb)
        if do_e:
            eissue(j + 3, (b8 + 3) % ER)
        scale(b8, b)
        sissue(b8, b)

    for b8 in range(ER):
        step(b8, b8, first=(b8 < 2), do_g=True, do_e=True)

    def middle(g, carry):
        j0 = g * ER
        for b8 in range(ER):
            step(j0 + b8, b8, first=False, do_g=True, do_e=True)
        return carry

    lax.fori_loop(1, NCHUNK // ER - 1, middle, 0)

    j0 = NCHUNK - ER
    for b8 in range(ER):
        j = j0 + b8
        step(j, b8, first=False,
             do_g=(j + 2 < NCHUNK), do_e=(j + 3 < NCHUNK))
    swait((NCHUNK - 2) % ER, (NCHUNK - 2) % NBUF)
    swait((NCHUNK - 1) % ER, (NCHUNK - 1) % NBUF)

    plsc.subcore_barrier()

    pltpu.sync_copy(acc.at[pl.ds(s * TILE_STRIDE, TILE_ROWS)],
                    out_hbm.at[c, pl.ds(s * TILE_STRIDE, TILE_ROWS)])


@jax.jit
def _lap_sc(a_split, src3, dst3, val3):
    mesh = plsc.VectorSubcoreMesh(core_axis_name="c", subcore_axis_name="s")
    f = pl.kernel(
        _lap_body,
        out_type=jax.ShapeDtypeStruct((NC, N, HD), jnp.float32),
        mesh=mesh,
        scratch_types=(
            [
                pltpu.VMEM((ER, K), jnp.int32),
                pltpu.VMEM((ER, K), jnp.int32),
                pltpu.VMEM((ER, K), jnp.float32),
                pltpu.VMEM((64, HD), jnp.float32),
                pltpu.VMEM_SHARED((N, HD), jnp.float32),
                pltpu.VMEM_SHARED((N, HD), jnp.bfloat16),
            ]
            + [pltpu.VMEM((K, HD), jnp.bfloat16)] * NBUF
            + [pltpu.VMEM((K, HD), jnp.float32)] * NBUF
            + [pltpu.SemaphoreType.DMA] * (2 * NBUF + 6)
        ),
        compiler_params=pltpu.CompilerParams(use_tc_tiling_on_sc=False,
                                             needs_layout_passes=False),
    )
    return f(a_split, src3, dst3, val3)


NB = 10
R = N // NB

_PERM = np.empty(D, np.int32)
for _h in range(2):
    for _g in range(2):
        _base = 64 * _h + 32 * _g
        for _t in range(16):
            _PERM[_base + 2 * _t] = _base + _t
            _PERM[_base + 2 * _t + 1] = _base + 16 + _t


def _elu(x):
    return jnp.where(x > 0, x, jnp.exp(jnp.minimum(x, 0.0)) - 1.0)


def _bn_consts(stats_ref, g_ref, be_ref):
    mean = stats_ref[0:1, :] * (1.0 / N)
    ex2 = stats_ref[1:2, :] * (1.0 / N)
    var = ex2 - mean * mean
    sc = g_ref[0:1, :] * jax.lax.rsqrt(var + 1e-5)
    sh = be_ref[0:1, :] - mean * sc
    return sc, sh


def _acc_stats(stats_ref, blk, i):
    @pl.when(i == 0)
    def _():
        stats_ref[...] = jnp.zeros_like(stats_ref)

    stats_ref[0:1, :] += jnp.sum(blk, axis=0, keepdims=True)
    stats_ref[1:2, :] += jnp.sum(blk * blk, axis=0, keepdims=True)


def _pre_body(inp_ref, w_ref, b_ref, x_ref, a_ref, stats_ref):
    x = jnp.dot(inp_ref[...], w_ref[...],
                preferred_element_type=jnp.float32) + b_ref[0:1, :]
    x_ref[...] = x
    a = _elu(x)
    a_ref[...] = a
    stats_ref[...] = jnp.zeros_like(stats_ref)
    stats_ref[0:1, :] = jnp.sum(a, axis=0, keepdims=True)
    stats_ref[1:2, :] = jnp.sum(a * a, axis=0, keepdims=True)


@jax.jit
def _pre(inp, w1, b1):
    return pl.pallas_call(
        _pre_body,
        out_shape=[
            jax.ShapeDtypeStruct((N, D), jnp.float32),
            jax.ShapeDtypeStruct((N, D), jnp.float32),
            jax.ShapeDtypeStruct((8, D), jnp.float32),
        ],
    )(inp, w1, b1)


def _make_lap_step(with_res):
    def body(*refs):
        if with_res:
            (a_ref, p_ref, res_ref, stats_ref, gt, gb, bet, beb,
             wt, wb, bv, x_ref, a_out, stats_out, opst) = refs
        else:
            (a_ref, p_ref, stats_ref, gt, gb, bet, beb,
             wt, wb, bv, x_ref, a_out, stats_out, opst) = refs
        ph = pl.program_id(0)
        i = pl.program_id(1)
        op = jnp.concatenate([p_ref[0], p_ref[1]], axis=1)

        @pl.when(ph == 0)
        def _():
            _acc_stats(opst, op, i)
            @pl.when(i == 0)
            def _():
                x_ref[...] = jnp.zeros_like(x_ref)
                a_out[...] = jnp.zeros_like(a_out)
                stats_out[...] = jnp.zeros_like(stats_out)

        @pl.when(ph == 1)
        def _():
            sct, sht = _bn_consts(stats_ref, gt, bet)
            scb, shb = _bn_consts(opst, gb, beb)
            y = jnp.dot(a_ref[...] * sct + sht, wt[...],
                        preferred_element_type=jnp.float32)
            y += jnp.dot(op * scb + shb, wb[...],
                         preferred_element_type=jnp.float32)
            y += bv[0:1, :]
            if with_res:
                y += res_ref[...]
            x_ref[...] = y
            a = _elu(y)
            a_out[...] = a
            _acc_stats(stats_out, a, i)

    return body


def _make_lap_call(with_res):
    body = _make_lap_step(with_res)
    zero2 = lambda ph, i: (0, 0)
    mov2 = lambda ph, i: (i * ph, 0)
    res_spec = [pl.BlockSpec((R, D), mov2)] if with_res else []

    @jax.jit
    def call(a, p, res, stats, gt, gb, bet, beb, wt, wb, bv):
        args = [a, p] + ([res] if with_res else []) + [
            stats, gt, gb, bet, beb, wt, wb, bv]
        return pl.pallas_call(
            body,
            grid=(2, NB),
            in_specs=[
                pl.BlockSpec((R, D), mov2),
                pl.BlockSpec((NC, R, HD), lambda ph, i: (0, i, 0)),
            ] + res_spec + [
                pl.BlockSpec((8, D), zero2),
                pl.BlockSpec((1, D), zero2),
                pl.BlockSpec((1, D), zero2),
                pl.BlockSpec((1, D), zero2),
                pl.BlockSpec((1, D), zero2),
                pl.BlockSpec((D, D), zero2),
                pl.BlockSpec((D, D), zero2),
                pl.BlockSpec((1, D), zero2),
            ],
            out_specs=[
                pl.BlockSpec((R, D), mov2),
                pl.BlockSpec((R, D), mov2),
                pl.BlockSpec((8, D), zero2),
            ],
            out_shape=[
                jax.ShapeDtypeStruct((N, D), jnp.float32),
                jax.ShapeDtypeStruct((N, D), jnp.float32),
                jax.ShapeDtypeStruct((8, D), jnp.float32),
            ],
            scratch_shapes=[pltpu.VMEM((8, D), jnp.float32)],
        )(*args)

    return call


def _make_avg_step(with_res):
    def body(*refs):
        if with_res:
            (a_ref, res_ref, stats_ref, gt, bet, beb,
             wt, wb, bv, x_ref, a_out, stats_out) = refs
        else:
            (a_ref, stats_ref, gt, bet, beb,
             wt, wb, bv, x_ref, a_out, stats_out) = refs
        i = pl.program_id(0)
        sct, sht = _bn_consts(stats_ref, gt, bet)
        y = jnp.dot(a_ref[...] * sct + sht, wt[...],
                    preferred_element_type=jnp.float32)
        y += jnp.dot(beb[0:1, :], wb[...],
                     preferred_element_type=jnp.float32)
        y += bv[0:1, :]
        if with_res:
            y += res_ref[...]
        x_ref[...] = y
        a = _elu(y)
        a_out[...] = a
        _acc_stats(stats_out, a, i)

    return body


def _make_avg_call(with_res):
    body = _make_avg_step(with_res)
    zero1 = lambda i: (0, 0)
    mov1 = lambda i: (i, 0)
    res_spec = [pl.BlockSpec((R, D), mov1)] if with_res else []

    @jax.jit
    def call(a, res, stats, gt, bet, beb, wt, wb, bv):
        args = [a] + ([res] if with_res else []) + [
            stats, gt, bet, beb, wt, wb, bv]
        return pl.pallas_call(
            body,
            grid=(NB,),
            in_specs=[
                pl.BlockSpec((R, D), mov1),
            ] + res_spec + [
                pl.BlockSpec((8, D), zero1),
                pl.BlockSpec((1, D), zero1),
                pl.BlockSpec((1, D), zero1),
                pl.BlockSpec((1, D), zero1),
                pl.BlockSpec((D, D), zero1),
                pl.BlockSpec((D, D), zero1),
                pl.BlockSpec((1, D), zero1),
            ],
            out_specs=[
                pl.BlockSpec((R, D), mov1),
                pl.BlockSpec((R, D), mov1),
                pl.BlockSpec((8, D), zero1),
            ],
            out_shape=[
                jax.ShapeDtypeStruct((N, D), jnp.float32),
                jax.ShapeDtypeStruct((N, D), jnp.float32),
                jax.ShapeDtypeStruct((8, D), jnp.float32),
            ],
        )(*args)

    return call


_lap_step_nores = _make_lap_call(False)
_lap_step_res = _make_lap_call(True)
_avg_step_nores = _make_avg_call(False)
_avg_step_res = _make_avg_call(True)


def _final_body(a_ref, stats_ref, g_ref, be_ref, w_ref, b_ref, rep_ref,
                out_ref):
    sc, sh = _bn_consts(stats_ref, g_ref, be_ref)
    y = jnp.dot(a_ref[...] * sc + sh, w_ref[...],
                preferred_element_type=jnp.float32)
    out_ref[...] = y + b_ref[0:1, :] + rep_ref[...]


@jax.jit
def _final(a, stats, g, be, w2, b2, rep):
    return pl.pallas_call(
        _final_body,
        out_shape=jax.ShapeDtypeStruct((N, 120), jnp.float32),
    )(a, stats, g, be, w2, b2, rep)


def kernel(inputs, mask, L_indices, L_values, W1, b1,
           rn_g0, rn_be0, rn_W0, rn_b0,
           rn_g1, rn_be1, rn_W1, rn_b1,
           g2, be2, W2, b2):
    dst = L_indices[0].astype(jnp.int32)
    src = L_indices[1].astype(jnp.int32)
    val = L_values.astype(jnp.float32)
    pad = EPAD - E
    dst3 = jnp.pad(dst, (0, pad)).reshape(NS, NCHUNK, K)
    src3 = jnp.pad(src, (0, pad)).reshape(NS, NCHUNK, K)
    val3 = jnp.pad(val, (0, pad)).reshape(NS, NCHUNK, K)

    inp = inputs.reshape(N, 6)
    x, a, stats = _pre(inp, W1, b1.reshape(1, D))

    res = x
    for i in range(15):
        for j in range(2):
            g = (rn_g0, rn_be0, rn_W0, rn_b0) if j == 0 else \
                (rn_g1, rn_be1, rn_W1, rn_b1)
            gam, bet, w, bb = (t[i] for t in g)
            gt = gam[:D].reshape(1, D)
            gb = gam[D:].reshape(1, D)
            bt = bet[:D].reshape(1, D)
            bbt = bet[D:].reshape(1, D)
            wt = w[:D, :]
            wb = w[D:, :]
            bv = bb.reshape(1, D)
            r = res if j == 1 else None
            if i % 2 == 0:
                a_split = (a[:, _PERM].astype(jnp.bfloat16)
                           .reshape(N, NC, HD).transpose(1, 0, 2))
                p = _lap_sc(a_split, src3, dst3, val3)
                if j == 1:
                    x, a, stats = _lap_step_res(
                        a, p, r, stats, gt, gb, bt, bbt, wt, wb, bv)
                else:
                    x, a, stats = _lap_step_nores(
                        a, p, None, stats, gt, gb, bt, bbt, wt, wb, bv)
            else:
                if j == 1:
                    x, a, stats = _avg_step_res(
                        a, r, stats, gt, bt, bbt, wt, wb, bv)
                else:
                    x, a, stats = _avg_step_nores(
                        a, None, stats, gt, bt, bbt, wt, wb, bv)
            if j == 1:
                res = x
    rep = jnp.tile(inp[:, 3:6], (1, 40))
    out = _final(a, stats, g2.reshape(1, D), be2.reshape(1, D), W2,
                 b2.reshape(1, 120), rep)
    return out[None]
